# Initial kernel scaffold; baseline (speedup 1.0000x reference)
import jax
import jax.numpy as jnp
from jax.experimental import pallas as pl


def kernel(x, edge_index, batch, W1, b1, W2, b2, W3, b3, W4, b4, W5, b5):
    raise NotImplementedError("write your pallas kernel here")


# speedup vs baseline: 16.3261x; 16.3261x over previous
import functools

import jax
import jax.numpy as jnp
from jax import lax
from jax.experimental import pallas as pl
from jax.experimental.pallas import tpu as pltpu
from jax.experimental.pallas import tpu_sc as plsc

N = 10000
E = 320000
F = 128
G = 64
H2 = 64
C = 10

NC, NS, L = 2, 16, 16
NW = NC * NS
NP = 10240
EPW = E // NW
CH = 128
NCH = EPW // CH
TAIL = EPW - NCH * CH
RPS = 624
RTL = N - NS * RPS
DPS = NP // NS

RB = 1000
NRB = N // RB

_mesh = plsc.VectorSubcoreMesh(
    core_axis_name="c", subcore_axis_name="s", num_cores=NC, num_subcores=NS)


@functools.partial(
    pl.kernel,
    out_type=(jax.ShapeDtypeStruct((NP,), jnp.float32),
              jax.ShapeDtypeStruct((NP,), jnp.float32)),
    mesh=_mesh,
    scratch_types=dict(
        ones_v=pltpu.VMEM((CH,), jnp.float32),
        zero_v=pltpu.VMEM((DPS,), jnp.float32),
        idx_v=pltpu.VMEM((CH,), jnp.int32),
        idx_t=pltpu.VMEM((TAIL,), jnp.int32),
        deg_sh=pltpu.VMEM_SHARED((NP,), jnp.float32),
    ),
)
def _sc_deg(dst_h, deg0_out, deg1_out, ones_v, zero_v, idx_v, idx_t, deg_sh):
    cid = lax.axis_index("c")
    sid = lax.axis_index("s")
    w = sid * NC + cid
    base = w * EPW
    for k in range(CH // L):
        ones_v[pl.ds(k * L, L)] = jnp.ones((L,), jnp.float32)
    for k in range(DPS // L):
        zero_v[pl.ds(k * L, L)] = jnp.zeros((L,), jnp.float32)
    pltpu.sync_copy(zero_v, deg_sh.at[pl.ds(sid * DPS, DPS)])
    plsc.subcore_barrier()

    def body(i, carry):
        pltpu.sync_copy(dst_h.at[pl.ds(base + i * CH, CH)], idx_v)
        pltpu.sync_copy(ones_v, deg_sh.at[idx_v], add=True)
        return carry

    lax.fori_loop(0, NCH, body, 0)
    pltpu.sync_copy(dst_h.at[pl.ds(base + NCH * CH, TAIL)], idx_t)
    pltpu.sync_copy(ones_v.at[pl.ds(0, TAIL)], deg_sh.at[idx_t], add=True)
    plsc.subcore_barrier()

    @pl.when(cid == 0)
    def _():
        pltpu.sync_copy(deg_sh.at[pl.ds(sid * DPS, DPS)],
                        deg0_out.at[pl.ds(sid * DPS, DPS)])

    @pl.when(cid == 1)
    def _():
        pltpu.sync_copy(deg_sh.at[pl.ds(sid * DPS, DPS)],
                        deg1_out.at[pl.ds(sid * DPS, DPS)])


@functools.partial(
    pl.kernel,
    out_type=jax.ShapeDtypeStruct((NC, N, F), jnp.float32),
    mesh=_mesh,
    scratch_types=dict(
        sidx=pltpu.VMEM((CH,), jnp.int32),
        didx=pltpu.VMEM((CH,), jnp.int32),
        sidx_t=pltpu.VMEM((TAIL,), jnp.int32),
        didx_t=pltpu.VMEM((TAIL,), jnp.int32),
        rows_v=pltpu.VMEM((CH, F), jnp.float32),
        rows_t=pltpu.VMEM((TAIL, F), jnp.float32),
        acc_sh=pltpu.VMEM_SHARED((N, F), jnp.float32),
        sem=pltpu.SemaphoreType.DMA,
    ),
)
def _sc_scatter(y_h, src_h, dst_h, out_h,
                sidx, didx, sidx_t, didx_t, rows_v, rows_t, acc_sh, sem):
    cid = lax.axis_index("c")
    sid = lax.axis_index("s")
    w = sid * NC + cid
    base = w * EPW
    pltpu.sync_copy(y_h.at[pl.ds(sid * RPS, RPS)],
                    acc_sh.at[pl.ds(sid * RPS, RPS)])

    @pl.when(sid == NS - 1)
    def _():
        pltpu.sync_copy(y_h.at[pl.ds(NS * RPS, RTL)],
                        acc_sh.at[pl.ds(NS * RPS, RTL)])

    plsc.subcore_barrier()

    def body(i, carry):
        pltpu.sync_copy(src_h.at[pl.ds(base + i * CH, CH)], sidx)
        pltpu.sync_copy(dst_h.at[pl.ds(base + i * CH, CH)], didx)
        pltpu.async_copy(y_h.at[sidx], rows_v, sem).wait()
        pltpu.sync_copy(rows_v, acc_sh.at[didx], add=True)
        return carry

    lax.fori_loop(0, NCH, body, 0)
    pltpu.sync_copy(src_h.at[pl.ds(base + NCH * CH, TAIL)], sidx_t)
    pltpu.sync_copy(dst_h.at[pl.ds(base + NCH * CH, TAIL)], didx_t)
    pltpu.async_copy(y_h.at[sidx_t], rows_t, sem).wait()
    pltpu.sync_copy(rows_t, acc_sh.at[didx_t], add=True)
    plsc.subcore_barrier()
    pltpu.sync_copy(acc_sh.at[pl.ds(sid * RPS, RPS)],
                    out_h.at[cid, pl.ds(sid * RPS, RPS)])

    @pl.when(sid == NS - 1)
    def _():
        pltpu.sync_copy(acc_sh.at[pl.ds(NS * RPS, RTL)],
                        out_h.at[cid, pl.ds(NS * RPS, RTL)])


def _tc_pre_body(deg0_ref, deg1_ref, x_ref, w1_ref, y_ref, dinv_ref):
    deg = deg0_ref[...] + deg1_ref[...] + 1.0
    dinv = lax.rsqrt(deg)
    y = jnp.dot(x_ref[...], w1_ref[...], preferred_element_type=jnp.float32)
    y_ref[...] = y * dinv
    dinv_ref[...] = dinv


def _tc_pre(deg0, deg1, x, w1):
    return pl.pallas_call(
        _tc_pre_body,
        grid=(NRB,),
        in_specs=[
            pl.BlockSpec((RB, 1), lambda i: (i, 0)),
            pl.BlockSpec((RB, 1), lambda i: (i, 0)),
            pl.BlockSpec((RB, F), lambda i: (i, 0)),
            pl.BlockSpec((F, F), lambda i: (0, 0)),
        ],
        out_specs=[
            pl.BlockSpec((RB, F), lambda i: (i, 0)),
            pl.BlockSpec((RB, 1), lambda i: (i, 0)),
        ],
        out_shape=[
            jax.ShapeDtypeStruct((N, F), jnp.float32),
            jax.ShapeDtypeStruct((N, 1), jnp.float32),
        ],
    )(deg0, deg1, x, w1)


def _tc_mid_body(p_ref, y_ref, dinv_ref, b1_ref, w2_ref, y2_ref):
    acc = p_ref[0] + p_ref[1] - y_ref[...]
    h = acc * dinv_ref[...] + b1_ref[...]
    h = jnp.where(h >= 0, h, 0.01 * h)
    y2 = jnp.dot(h, w2_ref[...], preferred_element_type=jnp.float32)
    y2_ref[...] = y2 * dinv_ref[...]


def _tc_mid(parts, y1, dinv, b1, w2):
    return pl.pallas_call(
        _tc_mid_body,
        grid=(NRB,),
        in_specs=[
            pl.BlockSpec((NC, RB, F), lambda i: (0, i, 0)),
            pl.BlockSpec((RB, F), lambda i: (i, 0)),
            pl.BlockSpec((RB, 1), lambda i: (i, 0)),
            pl.BlockSpec((1, F), lambda i: (0, 0)),
            pl.BlockSpec((F, F), lambda i: (0, 0)),
        ],
        out_specs=pl.BlockSpec((RB, F), lambda i: (i, 0)),
        out_shape=jax.ShapeDtypeStruct((N, F), jnp.float32),
    )(parts, y1, dinv, b1, w2)


def _tc_post_body(q_ref, y2_ref, dinv_ref, b2_ref, batch_ref,
                  w3_ref, b3_ref, w4_ref, b4_ref, w5_ref, b5_ref,
                  out_ref, ssum, scnt):
    i = pl.program_id(0)
    acc = q_ref[0] + q_ref[1] - y2_ref[...]
    h = acc * dinv_ref[...] + b2_ref[...]
    h = jnp.where(h >= 0, h, 0.01 * h)
    gids = lax.broadcasted_iota(jnp.int32, (1, G), 1)
    oh = (batch_ref[...] == gids).astype(jnp.float32)
    dn = (((0,), (0,)), ((), ()))
    s_part = lax.dot_general(oh, h, dn, preferred_element_type=jnp.float32)
    c_part = lax.dot_general(oh, jnp.ones((RB, F), jnp.float32), dn,
                             preferred_element_type=jnp.float32)

    @pl.when(i == 0)
    def _():
        ssum[...] = s_part
        scnt[...] = c_part

    @pl.when(i > 0)
    def _():
        ssum[...] += s_part
        scnt[...] += c_part

    @pl.when(i == NRB - 1)
    def _():
        g = ssum[...] / jnp.maximum(scnt[...], 1.0)
        a = jnp.dot(g, w3_ref[...], preferred_element_type=jnp.float32)
        a = a + b3_ref[...]
        a = jnp.where(a >= 0, a, 0.01 * a)
        a = jnp.dot(a, w4_ref[...], preferred_element_type=jnp.float32)
        a = a + b4_ref[...]
        a = jnp.where(a >= 0, a, 0.01 * a)
        a = jnp.dot(a, w5_ref[...], preferred_element_type=jnp.float32)
        out_ref[...] = a + b5_ref[...]


def _tc_post(parts, y2, dinv, b2, batch2, w3, b3, w4, b4, w5, b5):
    return pl.pallas_call(
        _tc_post_body,
        grid=(NRB,),
        in_specs=[
            pl.BlockSpec((NC, RB, F), lambda i: (0, i, 0)),
            pl.BlockSpec((RB, F), lambda i: (i, 0)),
            pl.BlockSpec((RB, 1), lambda i: (i, 0)),
            pl.BlockSpec((1, F), lambda i: (0, 0)),
            pl.BlockSpec((RB, 1), lambda i: (i, 0)),
            pl.BlockSpec((F, H2), lambda i: (0, 0)),
            pl.BlockSpec((1, H2), lambda i: (0, 0)),
            pl.BlockSpec((H2, H2), lambda i: (0, 0)),
            pl.BlockSpec((1, H2), lambda i: (0, 0)),
            pl.BlockSpec((H2, C), lambda i: (0, 0)),
            pl.BlockSpec((1, C), lambda i: (0, 0)),
        ],
        out_specs=pl.BlockSpec((G, C), lambda i: (0, 0)),
        out_shape=jax.ShapeDtypeStruct((G, C), jnp.float32),
        scratch_shapes=[
            pltpu.VMEM((G, F), jnp.float32),
            pltpu.VMEM((G, F), jnp.float32),
        ],
    )(parts, y2, dinv, b2, batch2, w3, b3, w4, b4, w5, b5)


def kernel(x, edge_index, batch, W1, b1, W2, b2, W3, b3, W4, b4, W5, b5):
    src = edge_index[0]
    dst = edge_index[1]
    deg0, deg1 = _sc_deg(dst)
    y1, dinv = _tc_pre(deg0[:N].reshape(N, 1), deg1[:N].reshape(N, 1), x, W1)
    parts1 = _sc_scatter(y1, src, dst)
    y2 = _tc_mid(parts1, y1, dinv, b1.reshape(1, F), W2)
    parts2 = _sc_scatter(y2, src, dst)
    return _tc_post(parts2, y2, dinv, b2.reshape(1, F),
                    batch.reshape(N, 1).astype(jnp.int32),
                    W3, b3.reshape(1, H2), W4, b4.reshape(1, H2),
                    W5, b5.reshape(1, C))


# speedup vs baseline: 23.5744x; 1.4440x over previous
import functools

import jax
import jax.numpy as jnp
from jax import lax
from jax.experimental import pallas as pl
from jax.experimental.pallas import tpu as pltpu
from jax.experimental.pallas import tpu_sc as plsc

N = 10000
E = 320000
F = 128
G = 64
H2 = 64
C = 10

NC, NS, L = 2, 16, 16
NW = NC * NS
NP = 10240
EPW = E // NW
CH = 128
NCH = EPW // CH
TAIL = EPW - NCH * CH
RPS = 624
RTL = N - NS * RPS
DPS = NP // NS

RB = 1000
NRB = N // RB

_mesh = plsc.VectorSubcoreMesh(
    core_axis_name="c", subcore_axis_name="s", num_cores=NC, num_subcores=NS)


@functools.partial(
    pl.kernel,
    out_type=(jax.ShapeDtypeStruct((NP,), jnp.float32),
              jax.ShapeDtypeStruct((NP,), jnp.float32)),
    mesh=_mesh,
    scratch_types=dict(
        ones_v=pltpu.VMEM((CH,), jnp.float32),
        zero_v=pltpu.VMEM((DPS,), jnp.float32),
        idx_v=pltpu.VMEM((CH,), jnp.int32),
        idx_t=pltpu.VMEM((TAIL,), jnp.int32),
        deg_sh=pltpu.VMEM_SHARED((NP,), jnp.float32),
    ),
)
def _sc_deg(dst_h, deg0_out, deg1_out, ones_v, zero_v, idx_v, idx_t, deg_sh):
    cid = lax.axis_index("c")
    sid = lax.axis_index("s")
    w = sid * NC + cid
    base = w * EPW
    for k in range(CH // L):
        ones_v[pl.ds(k * L, L)] = jnp.ones((L,), jnp.float32)
    for k in range(DPS // L):
        zero_v[pl.ds(k * L, L)] = jnp.zeros((L,), jnp.float32)
    pltpu.sync_copy(zero_v, deg_sh.at[pl.ds(sid * DPS, DPS)])
    plsc.subcore_barrier()

    def body(i, carry):
        pltpu.sync_copy(dst_h.at[pl.ds(base + i * CH, CH)], idx_v)
        pltpu.sync_copy(ones_v, deg_sh.at[idx_v], add=True)
        return carry

    lax.fori_loop(0, NCH, body, 0)
    pltpu.sync_copy(dst_h.at[pl.ds(base + NCH * CH, TAIL)], idx_t)
    pltpu.sync_copy(ones_v.at[pl.ds(0, TAIL)], deg_sh.at[idx_t], add=True)
    plsc.subcore_barrier()

    @pl.when(cid == 0)
    def _():
        pltpu.sync_copy(deg_sh.at[pl.ds(sid * DPS, DPS)],
                        deg0_out.at[pl.ds(sid * DPS, DPS)])

    @pl.when(cid == 1)
    def _():
        pltpu.sync_copy(deg_sh.at[pl.ds(sid * DPS, DPS)],
                        deg1_out.at[pl.ds(sid * DPS, DPS)])


@functools.partial(
    pl.kernel,
    out_type=jax.ShapeDtypeStruct((NC, N, F), jnp.float32),
    mesh=_mesh,
    scratch_types=dict(
        sidx0=pltpu.VMEM((CH,), jnp.int32),
        sidx1=pltpu.VMEM((CH,), jnp.int32),
        didx0=pltpu.VMEM((CH,), jnp.int32),
        didx1=pltpu.VMEM((CH,), jnp.int32),
        rows0=pltpu.VMEM((CH, F), jnp.float32),
        rows1=pltpu.VMEM((CH, F), jnp.float32),
        sidx_t=pltpu.VMEM((TAIL,), jnp.int32),
        didx_t=pltpu.VMEM((TAIL,), jnp.int32),
        rows_t=pltpu.VMEM((TAIL, F), jnp.float32),
        acc_sh=pltpu.VMEM_SHARED((N, F), jnp.float32),
        semg0=pltpu.SemaphoreType.DMA,
        semg1=pltpu.SemaphoreType.DMA,
        sems0=pltpu.SemaphoreType.DMA,
        sems1=pltpu.SemaphoreType.DMA,
    ),
)
def _sc_scatter(y_h, src_h, dst_h, out_h,
                sidx0, sidx1, didx0, didx1, rows0, rows1,
                sidx_t, didx_t, rows_t, acc_sh,
                semg0, semg1, sems0, sems1):
    cid = lax.axis_index("c")
    sid = lax.axis_index("s")
    w = sid * NC + cid
    base = w * EPW
    sidx = (sidx0, sidx1)
    didx = (didx0, didx1)
    rows = (rows0, rows1)
    semg = (semg0, semg1)
    sems = (sems0, sems1)
    pltpu.sync_copy(y_h.at[pl.ds(sid * RPS, RPS)],
                    acc_sh.at[pl.ds(sid * RPS, RPS)])

    @pl.when(sid == NS - 1)
    def _():
        pltpu.sync_copy(y_h.at[pl.ds(NS * RPS, RTL)],
                        acc_sh.at[pl.ds(NS * RPS, RTL)])

    plsc.subcore_barrier()

    def step(c, b):
        b1 = 1 - b

        @pl.when(c >= 2)
        def _():
            pltpu.make_async_copy(
                rows[b], acc_sh.at[didx[b]], sems[b]).wait()

        pltpu.sync_copy(src_h.at[pl.ds(base + c * CH, CH)], sidx[b])
        pltpu.sync_copy(dst_h.at[pl.ds(base + c * CH, CH)], didx[b])
        pltpu.async_copy(y_h.at[sidx[b]], rows[b], semg[b])

        @pl.when(c >= 1)
        def _():
            pltpu.make_async_copy(y_h.at[sidx[b1]], rows[b1], semg[b1]).wait()
            pltpu.async_copy(rows[b1], acc_sh.at[didx[b1]], sems[b1],
                             add=True)

    def body(p, carry):
        step(2 * p, 0)
        step(2 * p + 1, 1)
        return carry

    lax.fori_loop(0, NCH // 2, body, 0)
    last = NCH - 1
    pltpu.make_async_copy(y_h.at[sidx[1]], rows[1], semg[1]).wait()
    pltpu.async_copy(rows[1], acc_sh.at[didx[1]], sems[1], add=True)
    pltpu.make_async_copy(rows[0], acc_sh.at[didx[0]], sems[0]).wait()
    pltpu.make_async_copy(rows[1], acc_sh.at[didx[1]], sems[1]).wait()
    del last
    pltpu.sync_copy(src_h.at[pl.ds(base + NCH * CH, TAIL)], sidx_t)
    pltpu.sync_copy(dst_h.at[pl.ds(base + NCH * CH, TAIL)], didx_t)
    pltpu.async_copy(y_h.at[sidx_t], rows_t, semg0).wait()
    pltpu.sync_copy(rows_t, acc_sh.at[didx_t], add=True)
    plsc.subcore_barrier()
    pltpu.sync_copy(acc_sh.at[pl.ds(sid * RPS, RPS)],
                    out_h.at[cid, pl.ds(sid * RPS, RPS)])

    @pl.when(sid == NS - 1)
    def _():
        pltpu.sync_copy(acc_sh.at[pl.ds(NS * RPS, RTL)],
                        out_h.at[cid, pl.ds(NS * RPS, RTL)])


def _tc_pre_body(deg0_ref, deg1_ref, x_ref, w1_ref, y_ref, dinv_ref):
    deg = deg0_ref[...] + deg1_ref[...] + 1.0
    dinv = lax.rsqrt(deg)
    y = jnp.dot(x_ref[...], w1_ref[...], preferred_element_type=jnp.float32)
    y_ref[...] = y * dinv
    dinv_ref[...] = dinv


def _tc_pre(deg0, deg1, x, w1):
    return pl.pallas_call(
        _tc_pre_body,
        grid=(NRB,),
        in_specs=[
            pl.BlockSpec((RB, 1), lambda i: (i, 0)),
            pl.BlockSpec((RB, 1), lambda i: (i, 0)),
            pl.BlockSpec((RB, F), lambda i: (i, 0)),
            pl.BlockSpec((F, F), lambda i: (0, 0)),
        ],
        out_specs=[
            pl.BlockSpec((RB, F), lambda i: (i, 0)),
            pl.BlockSpec((RB, 1), lambda i: (i, 0)),
        ],
        out_shape=[
            jax.ShapeDtypeStruct((N, F), jnp.float32),
            jax.ShapeDtypeStruct((N, 1), jnp.float32),
        ],
    )(deg0, deg1, x, w1)


def _tc_mid_body(p_ref, y_ref, dinv_ref, b1_ref, w2_ref, y2_ref):
    acc = p_ref[0] + p_ref[1] - y_ref[...]
    h = acc * dinv_ref[...] + b1_ref[...]
    h = jnp.where(h >= 0, h, 0.01 * h)
    y2 = jnp.dot(h, w2_ref[...], preferred_element_type=jnp.float32)
    y2_ref[...] = y2 * dinv_ref[...]


def _tc_mid(parts, y1, dinv, b1, w2):
    return pl.pallas_call(
        _tc_mid_body,
        grid=(NRB,),
        in_specs=[
            pl.BlockSpec((NC, RB, F), lambda i: (0, i, 0)),
            pl.BlockSpec((RB, F), lambda i: (i, 0)),
            pl.BlockSpec((RB, 1), lambda i: (i, 0)),
            pl.BlockSpec((1, F), lambda i: (0, 0)),
            pl.BlockSpec((F, F), lambda i: (0, 0)),
        ],
        out_specs=pl.BlockSpec((RB, F), lambda i: (i, 0)),
        out_shape=jax.ShapeDtypeStruct((N, F), jnp.float32),
    )(parts, y1, dinv, b1, w2)


def _tc_post_body(q_ref, y2_ref, dinv_ref, b2_ref, batch_ref,
                  w3_ref, b3_ref, w4_ref, b4_ref, w5_ref, b5_ref,
                  out_ref, ssum, scnt):
    i = pl.program_id(0)
    acc = q_ref[0] + q_ref[1] - y2_ref[...]
    h = acc * dinv_ref[...] + b2_ref[...]
    h = jnp.where(h >= 0, h, 0.01 * h)
    gids = lax.broadcasted_iota(jnp.int32, (1, G), 1)
    oh = (batch_ref[...] == gids).astype(jnp.float32)
    dn = (((0,), (0,)), ((), ()))
    s_part = lax.dot_general(oh, h, dn, preferred_element_type=jnp.float32)
    c_part = lax.dot_general(oh, jnp.ones((RB, F), jnp.float32), dn,
                             preferred_element_type=jnp.float32)

    @pl.when(i == 0)
    def _():
        ssum[...] = s_part
        scnt[...] = c_part

    @pl.when(i > 0)
    def _():
        ssum[...] += s_part
        scnt[...] += c_part

    @pl.when(i == NRB - 1)
    def _():
        g = ssum[...] / jnp.maximum(scnt[...], 1.0)
        a = jnp.dot(g, w3_ref[...], preferred_element_type=jnp.float32)
        a = a + b3_ref[...]
        a = jnp.where(a >= 0, a, 0.01 * a)
        a = jnp.dot(a, w4_ref[...], preferred_element_type=jnp.float32)
        a = a + b4_ref[...]
        a = jnp.where(a >= 0, a, 0.01 * a)
        a = jnp.dot(a, w5_ref[...], preferred_element_type=jnp.float32)
        out_ref[...] = a + b5_ref[...]


def _tc_post(parts, y2, dinv, b2, batch2, w3, b3, w4, b4, w5, b5):
    return pl.pallas_call(
        _tc_post_body,
        grid=(NRB,),
        in_specs=[
            pl.BlockSpec((NC, RB, F), lambda i: (0, i, 0)),
            pl.BlockSpec((RB, F), lambda i: (i, 0)),
            pl.BlockSpec((RB, 1), lambda i: (i, 0)),
            pl.BlockSpec((1, F), lambda i: (0, 0)),
            pl.BlockSpec((RB, 1), lambda i: (i, 0)),
            pl.BlockSpec((F, H2), lambda i: (0, 0)),
            pl.BlockSpec((1, H2), lambda i: (0, 0)),
            pl.BlockSpec((H2, H2), lambda i: (0, 0)),
            pl.BlockSpec((1, H2), lambda i: (0, 0)),
            pl.BlockSpec((H2, C), lambda i: (0, 0)),
            pl.BlockSpec((1, C), lambda i: (0, 0)),
        ],
        out_specs=pl.BlockSpec((G, C), lambda i: (0, 0)),
        out_shape=jax.ShapeDtypeStruct((G, C), jnp.float32),
        scratch_shapes=[
            pltpu.VMEM((G, F), jnp.float32),
            pltpu.VMEM((G, F), jnp.float32),
        ],
    )(parts, y2, dinv, b2, batch2, w3, b3, w4, b4, w5, b5)


def kernel(x, edge_index, batch, W1, b1, W2, b2, W3, b3, W4, b4, W5, b5):
    src = edge_index[0]
    dst = edge_index[1]
    deg0, deg1 = _sc_deg(dst)
    y1, dinv = _tc_pre(deg0[:N].reshape(N, 1), deg1[:N].reshape(N, 1), x, W1)
    parts1 = _sc_scatter(y1, src, dst)
    y2 = _tc_mid(parts1, y1, dinv, b1.reshape(1, F), W2)
    parts2 = _sc_scatter(y2, src, dst)
    return _tc_post(parts2, y2, dinv, b2.reshape(1, F),
                    batch.reshape(N, 1).astype(jnp.int32),
                    W3, b3.reshape(1, H2), W4, b4.reshape(1, H2),
                    W5, b5.reshape(1, C))


# speedup vs baseline: 32.7159x; 1.3878x over previous
import functools

import jax
import jax.numpy as jnp
from jax import lax
from jax.experimental import pallas as pl
from jax.experimental.pallas import tpu as pltpu
from jax.experimental.pallas import tpu_sc as plsc

N = 10000
E = 320000
F = 128
G = 64
H2 = 64
C = 10

NC, NS, L = 2, 16, 16
NW = NC * NS
NP = 10240
CH = 125
ECH = E // CH
CPT = ECH // NW
HPT = CPT // 2
RPS = 624
RTL = N - NS * RPS
DPS = NP // NS

RB = 1000
NRB = N // RB

_mesh = plsc.VectorSubcoreMesh(
    core_axis_name="c", subcore_axis_name="s", num_cores=NC, num_subcores=NS)


@functools.partial(
    pl.kernel,
    out_type=(jax.ShapeDtypeStruct((NP,), jnp.float32),
              jax.ShapeDtypeStruct((NP,), jnp.float32)),
    mesh=_mesh,
    scratch_types=dict(
        ones_v=pltpu.VMEM((128,), jnp.float32),
        zero_v=pltpu.VMEM((DPS,), jnp.float32),
        didx_v=pltpu.VMEM((CPT, CH), jnp.int32),
        deg_sh=pltpu.VMEM_SHARED((NP,), jnp.float32),
        sem=pltpu.SemaphoreType.DMA,
    ),
)
def _sc_deg(dst2_h, deg0_out, deg1_out, ones_v, zero_v, didx_v, deg_sh, sem):
    cid = lax.axis_index("c")
    sid = lax.axis_index("s")
    w = sid * NC + cid
    for k in range(128 // L):
        ones_v[pl.ds(k * L, L)] = jnp.ones((L,), jnp.float32)
    for k in range(DPS // L):
        zero_v[pl.ds(k * L, L)] = jnp.zeros((L,), jnp.float32)
    pltpu.sync_copy(dst2_h.at[pl.ds(w * CPT, CPT)], didx_v)
    pltpu.sync_copy(zero_v, deg_sh.at[pl.ds(sid * DPS, DPS)])
    plsc.subcore_barrier()

    ones = ones_v.at[pl.ds(0, CH)]

    def fire(c, carry):
        pltpu.async_copy(ones, deg_sh.at[didx_v.at[c]], sem, add=True)
        return carry

    lax.fori_loop(0, CPT, fire, 0)

    def drain(c, carry):
        pltpu.make_async_copy(ones, deg_sh.at[didx_v.at[c]], sem).wait()
        return carry

    lax.fori_loop(0, CPT, drain, 0)
    plsc.subcore_barrier()

    @pl.when(cid == 0)
    def _():
        pltpu.sync_copy(deg_sh.at[pl.ds(sid * DPS, DPS)],
                        deg0_out.at[pl.ds(sid * DPS, DPS)])

    @pl.when(cid == 1)
    def _():
        pltpu.sync_copy(deg_sh.at[pl.ds(sid * DPS, DPS)],
                        deg1_out.at[pl.ds(sid * DPS, DPS)])


@functools.partial(
    pl.kernel,
    out_type=jax.ShapeDtypeStruct((NC, N, F), jnp.float32),
    mesh=_mesh,
    scratch_types=dict(
        sidx_v=pltpu.VMEM((HPT, CH), jnp.int32),
        didx_v=pltpu.VMEM((HPT, CH), jnp.int32),
        rows0=pltpu.VMEM((CH, F), jnp.float32),
        rows1=pltpu.VMEM((CH, F), jnp.float32),
        acc_sh=pltpu.VMEM_SHARED((N, F), jnp.float32),
        semi=pltpu.SemaphoreType.DMA,
        semg0=pltpu.SemaphoreType.DMA,
        semg1=pltpu.SemaphoreType.DMA,
        sems0=pltpu.SemaphoreType.DMA,
        sems1=pltpu.SemaphoreType.DMA,
    ),
)
def _sc_scatter(y_h, src2_h, dst2_h, out_h,
                sidx_v, didx_v, rows0, rows1, acc_sh,
                semi, semg0, semg1, sems0, sems1):
    cid = lax.axis_index("c")
    sid = lax.axis_index("s")
    w = sid * NC + cid
    rows = (rows0, rows1)
    semg = (semg0, semg1)
    sems = (sems0, sems1)

    pltpu.sync_copy(src2_h.at[pl.ds(w * CPT, HPT)], sidx_v)
    pltpu.sync_copy(dst2_h.at[pl.ds(w * CPT, HPT)], didx_v)
    pltpu.async_copy(y_h.at[pl.ds(sid * RPS, RPS)],
                     acc_sh.at[pl.ds(sid * RPS, RPS)], semi)

    @pl.when(sid == NS - 1)
    def _():
        pltpu.sync_copy(y_h.at[pl.ds(NS * RPS, RTL)],
                        acc_sh.at[pl.ds(NS * RPS, RTL)])

    for b in range(2):
        pltpu.async_copy(y_h.at[sidx_v.at[b]], rows[b], semg[b])
    pltpu.make_async_copy(y_h.at[pl.ds(sid * RPS, RPS)],
                          acc_sh.at[pl.ds(sid * RPS, RPS)], semi).wait()
    plsc.subcore_barrier()

    def run_phase(h):
        def body(p, carry):
            for b in range(2):
                c = 2 * p + b

                @pl.when(c >= 2)
                def _():
                    pltpu.make_async_copy(
                        rows[b], acc_sh.at[didx_v.at[c - 2]], sems[b]).wait()
                    pltpu.async_copy(y_h.at[sidx_v.at[c]], rows[b], semg[b])

                @pl.when(c >= 1)
                def _():
                    b1 = 1 - b
                    pltpu.make_async_copy(y_h.at[sidx_v.at[c - 1]], rows[b1],
                                          semg[b1]).wait()
                    pltpu.async_copy(rows[b1], acc_sh.at[didx_v.at[c - 1]],
                                     sems[b1], add=True)
            return carry

        lax.fori_loop(0, HPT // 2, body, 0)
        last = HPT - 1
        pltpu.make_async_copy(y_h.at[sidx_v.at[last]], rows[1],
                              semg[1]).wait()
        pltpu.async_copy(rows[1], acc_sh.at[didx_v.at[last]], sems[1],
                         add=True)
        pltpu.make_async_copy(rows[0], acc_sh.at[didx_v.at[last - 1]],
                              sems[0]).wait()
        pltpu.make_async_copy(rows[1], acc_sh.at[didx_v.at[last]],
                              sems[1]).wait()

    run_phase(0)
    pltpu.sync_copy(src2_h.at[pl.ds(w * CPT + HPT, HPT)], sidx_v)
    pltpu.sync_copy(dst2_h.at[pl.ds(w * CPT + HPT, HPT)], didx_v)
    for b in range(2):
        pltpu.async_copy(y_h.at[sidx_v.at[b]], rows[b], semg[b])
    run_phase(1)
    plsc.subcore_barrier()
    pltpu.sync_copy(acc_sh.at[pl.ds(sid * RPS, RPS)],
                    out_h.at[cid, pl.ds(sid * RPS, RPS)])

    @pl.when(sid == NS - 1)
    def _():
        pltpu.sync_copy(acc_sh.at[pl.ds(NS * RPS, RTL)],
                        out_h.at[cid, pl.ds(NS * RPS, RTL)])


def _tc_pre_body(deg0_ref, deg1_ref, x_ref, w1_ref, y_ref, dinv_ref):
    deg = deg0_ref[...] + deg1_ref[...] + 1.0
    dinv = lax.rsqrt(deg)
    y = jnp.dot(x_ref[...], w1_ref[...], preferred_element_type=jnp.float32)
    y_ref[...] = y * dinv
    dinv_ref[...] = dinv


def _tc_pre(deg0, deg1, x, w1):
    return pl.pallas_call(
        _tc_pre_body,
        grid=(NRB,),
        in_specs=[
            pl.BlockSpec((RB, 1), lambda i: (i, 0)),
            pl.BlockSpec((RB, 1), lambda i: (i, 0)),
            pl.BlockSpec((RB, F), lambda i: (i, 0)),
            pl.BlockSpec((F, F), lambda i: (0, 0)),
        ],
        out_specs=[
            pl.BlockSpec((RB, F), lambda i: (i, 0)),
            pl.BlockSpec((RB, 1), lambda i: (i, 0)),
        ],
        out_shape=[
            jax.ShapeDtypeStruct((N, F), jnp.float32),
            jax.ShapeDtypeStruct((N, 1), jnp.float32),
        ],
    )(deg0, deg1, x, w1)


def _tc_mid_body(p_ref, y_ref, dinv_ref, b1_ref, w2_ref, y2_ref):
    acc = p_ref[0] + p_ref[1] - y_ref[...]
    h = acc * dinv_ref[...] + b1_ref[...]
    h = jnp.where(h >= 0, h, 0.01 * h)
    y2 = jnp.dot(h, w2_ref[...], preferred_element_type=jnp.float32)
    y2_ref[...] = y2 * dinv_ref[...]


def _tc_mid(parts, y1, dinv, b1, w2):
    return pl.pallas_call(
        _tc_mid_body,
        grid=(NRB,),
        in_specs=[
            pl.BlockSpec((NC, RB, F), lambda i: (0, i, 0)),
            pl.BlockSpec((RB, F), lambda i: (i, 0)),
            pl.BlockSpec((RB, 1), lambda i: (i, 0)),
            pl.BlockSpec((1, F), lambda i: (0, 0)),
            pl.BlockSpec((F, F), lambda i: (0, 0)),
        ],
        out_specs=pl.BlockSpec((RB, F), lambda i: (i, 0)),
        out_shape=jax.ShapeDtypeStruct((N, F), jnp.float32),
    )(parts, y1, dinv, b1, w2)


def _tc_post_body(q_ref, y2_ref, dinv_ref, b2_ref, batch_ref,
                  w3_ref, b3_ref, w4_ref, b4_ref, w5_ref, b5_ref,
                  out_ref, ssum, scnt):
    i = pl.program_id(0)
    acc = q_ref[0] + q_ref[1] - y2_ref[...]
    h = acc * dinv_ref[...] + b2_ref[...]
    h = jnp.where(h >= 0, h, 0.01 * h)
    gids = lax.broadcasted_iota(jnp.int32, (1, G), 1)
    oh = (batch_ref[...] == gids).astype(jnp.float32)
    dn = (((0,), (0,)), ((), ()))
    s_part = lax.dot_general(oh, h, dn, preferred_element_type=jnp.float32)
    c_part = lax.dot_general(oh, jnp.ones((RB, F), jnp.float32), dn,
                             preferred_element_type=jnp.float32)

    @pl.when(i == 0)
    def _():
        ssum[...] = s_part
        scnt[...] = c_part

    @pl.when(i > 0)
    def _():
        ssum[...] += s_part
        scnt[...] += c_part

    @pl.when(i == NRB - 1)
    def _():
        g = ssum[...] / jnp.maximum(scnt[...], 1.0)
        a = jnp.dot(g, w3_ref[...], preferred_element_type=jnp.float32)
        a = a + b3_ref[...]
        a = jnp.where(a >= 0, a, 0.01 * a)
        a = jnp.dot(a, w4_ref[...], preferred_element_type=jnp.float32)
        a = a + b4_ref[...]
        a = jnp.where(a >= 0, a, 0.01 * a)
        a = jnp.dot(a, w5_ref[...], preferred_element_type=jnp.float32)
        out_ref[...] = a + b5_ref[...]


def _tc_post(parts, y2, dinv, b2, batch2, w3, b3, w4, b4, w5, b5):
    return pl.pallas_call(
        _tc_post_body,
        grid=(NRB,),
        in_specs=[
            pl.BlockSpec((NC, RB, F), lambda i: (0, i, 0)),
            pl.BlockSpec((RB, F), lambda i: (i, 0)),
            pl.BlockSpec((RB, 1), lambda i: (i, 0)),
            pl.BlockSpec((1, F), lambda i: (0, 0)),
            pl.BlockSpec((RB, 1), lambda i: (i, 0)),
            pl.BlockSpec((F, H2), lambda i: (0, 0)),
            pl.BlockSpec((1, H2), lambda i: (0, 0)),
            pl.BlockSpec((H2, H2), lambda i: (0, 0)),
            pl.BlockSpec((1, H2), lambda i: (0, 0)),
            pl.BlockSpec((H2, C), lambda i: (0, 0)),
            pl.BlockSpec((1, C), lambda i: (0, 0)),
        ],
        out_specs=pl.BlockSpec((G, C), lambda i: (0, 0)),
        out_shape=jax.ShapeDtypeStruct((G, C), jnp.float32),
        scratch_shapes=[
            pltpu.VMEM((G, F), jnp.float32),
            pltpu.VMEM((G, F), jnp.float32),
        ],
    )(parts, y2, dinv, b2, batch2, w3, b3, w4, b4, w5, b5)


def kernel(x, edge_index, batch, W1, b1, W2, b2, W3, b3, W4, b4, W5, b5):
    src = edge_index[0].reshape(ECH, CH)
    dst = edge_index[1].reshape(ECH, CH)
    deg0, deg1 = _sc_deg(dst)
    y1, dinv = _tc_pre(deg0[:N].reshape(N, 1), deg1[:N].reshape(N, 1), x, W1)
    parts1 = _sc_scatter(y1, src, dst)
    y2 = _tc_mid(parts1, y1, dinv, b1.reshape(1, F), W2)
    parts2 = _sc_scatter(y2, src, dst)
    return _tc_post(parts2, y2, dinv, b2.reshape(1, F),
                    batch.reshape(N, 1).astype(jnp.int32),
                    W3, b3.reshape(1, H2), W4, b4.reshape(1, H2),
                    W5, b5.reshape(1, C))


# speedup vs baseline: 32.7187x; 1.0001x over previous
import functools

import jax
import jax.numpy as jnp
from jax import lax
from jax.experimental import pallas as pl
from jax.experimental.pallas import tpu as pltpu
from jax.experimental.pallas import tpu_sc as plsc

N = 10000
E = 320000
F = 128
G = 64
H2 = 64
C = 10

NC, NS, L = 2, 16, 16
NW = NC * NS
NP = 10240
CH = 125
ECH = E // CH
CPT = ECH // NW
HPT = CPT // 2
RPS = 624
RTL = N - NS * RPS
DPS = NP // NS

RB = 1000
NRB = N // RB

_mesh = plsc.VectorSubcoreMesh(
    core_axis_name="c", subcore_axis_name="s", num_cores=NC, num_subcores=NS)


@functools.partial(
    pl.kernel,
    out_type=(jax.ShapeDtypeStruct((NP,), jnp.float32),
              jax.ShapeDtypeStruct((NP,), jnp.float32)),
    mesh=_mesh,
    scratch_types=dict(
        ones_v=pltpu.VMEM((128,), jnp.float32),
        zero_v=pltpu.VMEM((DPS,), jnp.float32),
        didx_v=pltpu.VMEM((CPT, CH), jnp.int32),
        deg_sh=pltpu.VMEM_SHARED((NP,), jnp.float32),
        sem=pltpu.SemaphoreType.DMA,
    ),
)
def _sc_deg(dst2_h, deg0_out, deg1_out, ones_v, zero_v, didx_v, deg_sh, sem):
    cid = lax.axis_index("c")
    sid = lax.axis_index("s")
    w = sid * NC + cid
    for k in range(128 // L):
        ones_v[pl.ds(k * L, L)] = jnp.ones((L,), jnp.float32)
    for k in range(DPS // L):
        zero_v[pl.ds(k * L, L)] = jnp.zeros((L,), jnp.float32)
    pltpu.sync_copy(dst2_h.at[pl.ds(w * CPT, CPT)], didx_v)
    pltpu.sync_copy(zero_v, deg_sh.at[pl.ds(sid * DPS, DPS)])
    plsc.subcore_barrier()

    ones = ones_v.at[pl.ds(0, CH)]

    def fire(c, carry):
        pltpu.async_copy(ones, deg_sh.at[didx_v.at[c]], sem, add=True)
        return carry

    lax.fori_loop(0, CPT, fire, 0)

    def drain(c, carry):
        pltpu.make_async_copy(ones, deg_sh.at[didx_v.at[c]], sem).wait()
        return carry

    lax.fori_loop(0, CPT, drain, 0)
    plsc.subcore_barrier()

    @pl.when(cid == 0)
    def _():
        pltpu.sync_copy(deg_sh.at[pl.ds(sid * DPS, DPS)],
                        deg0_out.at[pl.ds(sid * DPS, DPS)])

    @pl.when(cid == 1)
    def _():
        pltpu.sync_copy(deg_sh.at[pl.ds(sid * DPS, DPS)],
                        deg1_out.at[pl.ds(sid * DPS, DPS)])


@functools.partial(
    pl.kernel,
    out_type=jax.ShapeDtypeStruct((NC, N, F), jnp.float32),
    mesh=_mesh,
    scratch_types=dict(
        sidx_v=pltpu.VMEM((HPT, CH), jnp.int32),
        didx_v=pltpu.VMEM((HPT, CH), jnp.int32),
        rows0=pltpu.VMEM((CH, F), jnp.float32),
        rows1=pltpu.VMEM((CH, F), jnp.float32),
        acc_sh=pltpu.VMEM_SHARED((N, F), jnp.float32),
        semi=pltpu.SemaphoreType.DMA,
        semg0=pltpu.SemaphoreType.DMA,
        semg1=pltpu.SemaphoreType.DMA,
        sems0=pltpu.SemaphoreType.DMA,
        sems1=pltpu.SemaphoreType.DMA,
    ),
)
def _sc_scatter(y_h, src2_h, dst2_h, out_h,
                sidx_v, didx_v, rows0, rows1, acc_sh,
                semi, semg0, semg1, sems0, sems1):
    cid = lax.axis_index("c")
    sid = lax.axis_index("s")
    w = sid * NC + cid
    rows = (rows0, rows1)
    semg = (semg0, semg1)
    sems = (sems0, sems1)

    def sidx(c):
        return sidx_v.at[c]

    pltpu.sync_copy(src2_h.at[pl.ds(w * CPT, HPT)], sidx_v)
    pltpu.sync_copy(dst2_h.at[pl.ds(w * CPT, HPT)], didx_v)
    pltpu.async_copy(y_h.at[pl.ds(sid * RPS, RPS)],
                     acc_sh.at[pl.ds(sid * RPS, RPS)], semi)

    @pl.when(sid == NS - 1)
    def _():
        pltpu.sync_copy(y_h.at[pl.ds(NS * RPS, RTL)],
                        acc_sh.at[pl.ds(NS * RPS, RTL)])

    for b in range(2):
        pltpu.async_copy(y_h.at[sidx(b)], rows[b], semg[b])
    pltpu.make_async_copy(y_h.at[pl.ds(sid * RPS, RPS)],
                          acc_sh.at[pl.ds(sid * RPS, RPS)], semi).wait()
    plsc.subcore_barrier()

    def run_phase(h):
        def body(p, carry):
            for b in range(2):
                c = 2 * p + b

                @pl.when(c >= 2)
                def _():
                    pltpu.make_async_copy(
                        rows[b], acc_sh.at[didx_v.at[c - 2]], sems[b]).wait()
                    pltpu.async_copy(y_h.at[sidx(c)], rows[b], semg[b])

                @pl.when(c >= 1)
                def _():
                    b1 = 1 - b
                    pltpu.make_async_copy(y_h.at[sidx(c - 1)], rows[b1],
                                          semg[b1]).wait()
                    pltpu.async_copy(rows[b1], acc_sh.at[didx_v.at[c - 1]],
                                     sems[b1], add=True)
            return carry

        lax.fori_loop(0, HPT // 2, body, 0)
        last = HPT - 1
        pltpu.make_async_copy(y_h.at[sidx(last)], rows[1],
                              semg[1]).wait()
        pltpu.async_copy(rows[1], acc_sh.at[didx_v.at[last]], sems[1],
                         add=True)
        pltpu.make_async_copy(rows[0], acc_sh.at[didx_v.at[last - 1]],
                              sems[0]).wait()
        pltpu.make_async_copy(rows[1], acc_sh.at[didx_v.at[last]],
                              sems[1]).wait()

    run_phase(0)
    pltpu.sync_copy(src2_h.at[pl.ds(w * CPT + HPT, HPT)], sidx_v)
    pltpu.sync_copy(dst2_h.at[pl.ds(w * CPT + HPT, HPT)], didx_v)
    for b in range(2):
        pltpu.async_copy(y_h.at[sidx(b)], rows[b], semg[b])
    run_phase(1)
    plsc.subcore_barrier()
    pltpu.sync_copy(acc_sh.at[pl.ds(sid * RPS, RPS)],
                    out_h.at[cid, pl.ds(sid * RPS, RPS)])

    @pl.when(sid == NS - 1)
    def _():
        pltpu.sync_copy(acc_sh.at[pl.ds(NS * RPS, RTL)],
                        out_h.at[cid, pl.ds(NS * RPS, RTL)])


def _tc_pre_body(deg0_ref, deg1_ref, x_ref, w1_ref, y_ref, dinv_ref):
    deg = deg0_ref[...] + deg1_ref[...] + 1.0
    dinv = lax.rsqrt(deg)
    y = jnp.dot(x_ref[...], w1_ref[...], preferred_element_type=jnp.float32)
    y_ref[...] = y * dinv
    dinv_ref[...] = dinv


def _tc_pre(deg0, deg1, x, w1):
    return pl.pallas_call(
        _tc_pre_body,
        grid=(NRB,),
        in_specs=[
            pl.BlockSpec((RB, 1), lambda i: (i, 0)),
            pl.BlockSpec((RB, 1), lambda i: (i, 0)),
            pl.BlockSpec((RB, F), lambda i: (i, 0)),
            pl.BlockSpec((F, F), lambda i: (0, 0)),
        ],
        out_specs=[
            pl.BlockSpec((RB, F), lambda i: (i, 0)),
            pl.BlockSpec((RB, 1), lambda i: (i, 0)),
        ],
        out_shape=[
            jax.ShapeDtypeStruct((N, F), jnp.float32),
            jax.ShapeDtypeStruct((N, 1), jnp.float32),
        ],
    )(deg0, deg1, x, w1)


def _tc_mid_body(p_ref, y_ref, dinv_ref, b1_ref, w2_ref, y2_ref):
    acc = p_ref[0] + p_ref[1] - y_ref[...]
    h = acc * dinv_ref[...] + b1_ref[...]
    h = jnp.where(h >= 0, h, 0.01 * h)
    y2 = jnp.dot(h, w2_ref[...], preferred_element_type=jnp.float32)
    y2_ref[...] = y2 * dinv_ref[...]


def _tc_mid(parts, y1, dinv, b1, w2):
    return pl.pallas_call(
        _tc_mid_body,
        grid=(NRB,),
        in_specs=[
            pl.BlockSpec((NC, RB, F), lambda i: (0, i, 0)),
            pl.BlockSpec((RB, F), lambda i: (i, 0)),
            pl.BlockSpec((RB, 1), lambda i: (i, 0)),
            pl.BlockSpec((1, F), lambda i: (0, 0)),
            pl.BlockSpec((F, F), lambda i: (0, 0)),
        ],
        out_specs=pl.BlockSpec((RB, F), lambda i: (i, 0)),
        out_shape=jax.ShapeDtypeStruct((N, F), jnp.float32),
    )(parts, y1, dinv, b1, w2)


def _tc_post_body(q_ref, y2_ref, dinv_ref, b2_ref, batch_ref,
                  w3_ref, b3_ref, w4_ref, b4_ref, w5_ref, b5_ref,
                  out_ref, ssum, scnt):
    i = pl.program_id(0)
    acc = q_ref[0] + q_ref[1] - y2_ref[...]
    h = acc * dinv_ref[...] + b2_ref[...]
    h = jnp.where(h >= 0, h, 0.01 * h)
    gids = lax.broadcasted_iota(jnp.int32, (1, G), 1)
    oh = (batch_ref[...] == gids).astype(jnp.float32)
    dn = (((0,), (0,)), ((), ()))
    s_part = lax.dot_general(oh, h, dn, preferred_element_type=jnp.float32)
    c_part = lax.dot_general(oh, jnp.ones((RB, F), jnp.float32), dn,
                             preferred_element_type=jnp.float32)

    @pl.when(i == 0)
    def _():
        ssum[...] = s_part
        scnt[...] = c_part

    @pl.when(i > 0)
    def _():
        ssum[...] += s_part
        scnt[...] += c_part

    @pl.when(i == NRB - 1)
    def _():
        g = ssum[...] / jnp.maximum(scnt[...], 1.0)
        a = jnp.dot(g, w3_ref[...], preferred_element_type=jnp.float32)
        a = a + b3_ref[...]
        a = jnp.where(a >= 0, a, 0.01 * a)
        a = jnp.dot(a, w4_ref[...], preferred_element_type=jnp.float32)
        a = a + b4_ref[...]
        a = jnp.where(a >= 0, a, 0.01 * a)
        a = jnp.dot(a, w5_ref[...], preferred_element_type=jnp.float32)
        out_ref[...] = a + b5_ref[...]


def _tc_post(parts, y2, dinv, b2, batch2, w3, b3, w4, b4, w5, b5):
    return pl.pallas_call(
        _tc_post_body,
        grid=(NRB,),
        in_specs=[
            pl.BlockSpec((NC, RB, F), lambda i: (0, i, 0)),
            pl.BlockSpec((RB, F), lambda i: (i, 0)),
            pl.BlockSpec((RB, 1), lambda i: (i, 0)),
            pl.BlockSpec((1, F), lambda i: (0, 0)),
            pl.BlockSpec((RB, 1), lambda i: (i, 0)),
            pl.BlockSpec((F, H2), lambda i: (0, 0)),
            pl.BlockSpec((1, H2), lambda i: (0, 0)),
            pl.BlockSpec((H2, H2), lambda i: (0, 0)),
            pl.BlockSpec((1, H2), lambda i: (0, 0)),
            pl.BlockSpec((H2, C), lambda i: (0, 0)),
            pl.BlockSpec((1, C), lambda i: (0, 0)),
        ],
        out_specs=pl.BlockSpec((G, C), lambda i: (0, 0)),
        out_shape=jax.ShapeDtypeStruct((G, C), jnp.float32),
        scratch_shapes=[
            pltpu.VMEM((G, F), jnp.float32),
            pltpu.VMEM((G, F), jnp.float32),
        ],
    )(parts, y2, dinv, b2, batch2, w3, b3, w4, b4, w5, b5)


def kernel(x, edge_index, batch, W1, b1, W2, b2, W3, b3, W4, b4, W5, b5):
    src = edge_index[0].reshape(ECH, CH)
    dst = edge_index[1].reshape(ECH, CH)
    deg0, deg1 = _sc_deg(dst)
    y1, dinv = _tc_pre(deg0[:N].reshape(N, 1), deg1[:N].reshape(N, 1), x, W1)
    parts1 = _sc_scatter(y1, src, dst)
    y2 = _tc_mid(parts1, y1, dinv, b1.reshape(1, F), W2)
    parts2 = _sc_scatter(y2, src, dst)
    return _tc_post(parts2, y2, dinv, b2.reshape(1, F),
                    batch.reshape(N, 1).astype(jnp.int32),
                    W3, b3.reshape(1, H2), W4, b4.reshape(1, H2),
                    W5, b5.reshape(1, C))
